# baseline (speedup 1.0000x reference)
import numpy as np
import jax
import jax.numpy as jnp
from jax.experimental import pallas as pl
from jax.experimental.pallas import tpu as pltpu

BB = 1024
KS = 5
C1, C2 = 4, 8
PH1, PW1 = 12, 12
PH2, PW2 = 4, 4
NF1 = PH1 * PW1 * C1
NF2 = PH2 * PW2 * C2
F1 = 32
NC = 10


def _onehot_shift(num_out, num_in, parity):
    a = np.zeros((KS, num_out, num_in), np.float32)
    for k in range(KS):
        for p in range(num_out):
            a[k, p, 2 * p + parity + k] = 1.0
    return a


_A1 = np.stack([_onehot_shift(PH1, 28, d) for d in range(2)])
_A2 = np.stack([_onehot_shift(PH2, PW1, d) for d in range(2)])


def _dense_conv_weights(w1, w2):
    w1 = w1.reshape(KS, KS, C1)
    w2 = w2.reshape(KS, KS, C1, C2)
    w1d = jnp.einsum('dkpi,elqj,klc->deijpqc', _A1, _A1, w1,
                     ).reshape(4, 784, NF1).astype(jnp.bfloat16)
    w2d = jnp.einsum('dkpi,elqj,klmc->deijmpqc', _A2, _A2, w2,
                     ).reshape(4, NF1, NF2).astype(jnp.bfloat16)
    return w1d, w2d


def _fused_kernel(w1d_ref, b1t_ref, w2d_ref, b2t_ref, f1w_ref, f1b_ref,
                  f2w_ref, f2b_ref, x_ref, o_ref):
    x = x_ref[0]
    o1 = [jnp.dot(x, w1d_ref[i], preferred_element_type=jnp.float32)
          for i in range(4)]
    p1 = jnp.maximum(jnp.maximum(o1[0], o1[1]), jnp.maximum(o1[2], o1[3]))
    p1 = jnp.maximum(p1 + b1t_ref[...], 0.0).astype(jnp.bfloat16)

    o2 = [jnp.dot(p1, w2d_ref[i], preferred_element_type=jnp.float32)
          for i in range(4)]
    p2 = jnp.maximum(jnp.maximum(o2[0], o2[1]), jnp.maximum(o2[2], o2[3]))
    p2 = jnp.maximum(p2 + b2t_ref[...], 0.0)

    y1 = jnp.dot(p2, f1w_ref[...], preferred_element_type=jnp.float32)
    y1 = jnp.maximum(y1 + f1b_ref[...], 0.0)
    logits = jnp.dot(y1, f2w_ref[...],
                     preferred_element_type=jnp.float32) + f2b_ref[...]
    z = logits - jnp.max(logits, axis=1, keepdims=True)
    lse = jnp.log(jnp.sum(jnp.exp(z), axis=1, keepdims=True))
    o_ref[0] = z - lse


def kernel(conv1_w, conv1_b, conv2_w, conv2_b, fc1_w, fc1_b, fc2_w, fc2_b, x):
    n = x.shape[0]
    pad = (-n) % BB
    if pad:
        x = jnp.concatenate(
            [x, jnp.zeros((pad,) + x.shape[1:], x.dtype)], axis=0)
    nblk = x.shape[0] // BB
    x_b = x.reshape(nblk, BB, 784).astype(jnp.bfloat16)

    w1d, w2d = _dense_conv_weights(conv1_w, conv2_w)
    b1t = jnp.tile(conv1_b.reshape(1, C1), (1, PH1 * PW1))
    b2t = jnp.tile(conv2_b.reshape(1, C2), (1, PH2 * PW2))
    f1w = fc1_w.reshape(NF2, F1)

    out = pl.pallas_call(
        _fused_kernel,
        out_shape=jax.ShapeDtypeStruct((nblk, BB, NC), jnp.float32),
        grid=(nblk,),
        in_specs=[
            pl.BlockSpec((4, 784, NF1), lambda i: (0, 0, 0)),
            pl.BlockSpec((1, NF1), lambda i: (0, 0)),
            pl.BlockSpec((4, NF1, NF2), lambda i: (0, 0, 0)),
            pl.BlockSpec((1, NF2), lambda i: (0, 0)),
            pl.BlockSpec((NF2, F1), lambda i: (0, 0)),
            pl.BlockSpec((1, F1), lambda i: (0, 0)),
            pl.BlockSpec((F1, NC), lambda i: (0, 0)),
            pl.BlockSpec((1, NC), lambda i: (0, 0)),
            pl.BlockSpec((1, BB, 784), lambda i: (i, 0, 0)),
        ],
        out_specs=pl.BlockSpec((1, BB, NC), lambda i: (i, 0, 0)),
        compiler_params=pltpu.CompilerParams(
            dimension_semantics=("parallel",),
            vmem_limit_bytes=64 * 1024 * 1024),
    )(w1d, b1t, w2d, b2t, f1w, fc1_b, fc2_w, fc2_b, x_b)

    return out.reshape(nblk * BB, NC)[:n]


# speedup vs baseline: 1.3363x; 1.3363x over previous
import numpy as np
import jax
import jax.numpy as jnp
from jax.experimental import pallas as pl
from jax.experimental.pallas import tpu as pltpu

BB = 1024
KS = 5
C1, C2 = 4, 8
PH1, PW1 = 12, 12
PH2, PW2 = 4, 4
NF1 = PH1 * PW1 * C1
NF2 = PH2 * PW2 * C2
F1 = 32
NC = 10


def _onehot_shift(num_out, num_in, parity):
    a = np.zeros((KS, num_out, num_in), np.float32)
    for k in range(KS):
        for p in range(num_out):
            a[k, p, 2 * p + parity + k] = 1.0
    return a


_A1 = np.stack([_onehot_shift(PH1, 28, d) for d in range(2)])
_A2 = np.stack([_onehot_shift(PH2, PW1, d) for d in range(2)])


def _dense_conv_weights(w1, w2):
    w1 = w1.reshape(KS, KS, C1)
    w2 = w2.reshape(KS, KS, C1, C2)
    w1d = jnp.einsum('dkpi,elqj,klc->deijpqc', _A1, _A1, w1,
                     ).reshape(4, 784, NF1).astype(jnp.bfloat16)
    w2d = jnp.einsum('dkpi,elqj,klmc->deijmpqc', _A2, _A2, w2,
                     ).reshape(4, NF1, NF2).astype(jnp.bfloat16)
    return w1d, w2d


def _fused_kernel(w1d_ref, b1t_ref, w2d_ref, b2t_ref, f1w_ref, f1b_ref,
                  f2w_ref, f2b_ref, x_ref, o_ref):
    x = x_ref[0].astype(jnp.bfloat16).reshape(BB, 784)
    o1 = [jnp.dot(x, w1d_ref[i], preferred_element_type=jnp.float32)
          for i in range(4)]
    p1 = jnp.maximum(jnp.maximum(o1[0], o1[1]), jnp.maximum(o1[2], o1[3]))
    p1 = jnp.maximum(p1 + b1t_ref[...], 0.0).astype(jnp.bfloat16)

    o2 = [jnp.dot(p1, w2d_ref[i], preferred_element_type=jnp.float32)
          for i in range(4)]
    p2 = jnp.maximum(jnp.maximum(o2[0], o2[1]), jnp.maximum(o2[2], o2[3]))
    p2 = jnp.maximum(p2 + b2t_ref[...], 0.0)

    y1 = jnp.dot(p2, f1w_ref[...], preferred_element_type=jnp.float32)
    y1 = jnp.maximum(y1 + f1b_ref[...], 0.0)
    logits = jnp.dot(y1, f2w_ref[...],
                     preferred_element_type=jnp.float32) + f2b_ref[...]
    z = logits - jnp.max(logits, axis=1, keepdims=True)
    lse = jnp.log(jnp.sum(jnp.exp(z), axis=1, keepdims=True))
    o_ref[0] = z - lse


def kernel(conv1_w, conv1_b, conv2_w, conv2_b, fc1_w, fc1_b, fc2_w, fc2_b, x):
    n = x.shape[0]
    pad = (-n) % BB
    if pad:
        x = jnp.concatenate(
            [x, jnp.zeros((pad,) + x.shape[1:], x.dtype)], axis=0)
    nblk = x.shape[0] // BB
    x_b = x.reshape(nblk, BB, 28, 28)

    w1d, w2d = _dense_conv_weights(conv1_w, conv2_w)
    b1t = jnp.tile(conv1_b.reshape(1, C1), (1, PH1 * PW1))
    b2t = jnp.tile(conv2_b.reshape(1, C2), (1, PH2 * PW2))
    f1w = fc1_w.reshape(NF2, F1)

    out = pl.pallas_call(
        _fused_kernel,
        out_shape=jax.ShapeDtypeStruct((nblk, BB, NC), jnp.float32),
        grid=(nblk,),
        in_specs=[
            pl.BlockSpec((4, 784, NF1), lambda i: (0, 0, 0)),
            pl.BlockSpec((1, NF1), lambda i: (0, 0)),
            pl.BlockSpec((4, NF1, NF2), lambda i: (0, 0, 0)),
            pl.BlockSpec((1, NF2), lambda i: (0, 0)),
            pl.BlockSpec((NF2, F1), lambda i: (0, 0)),
            pl.BlockSpec((1, F1), lambda i: (0, 0)),
            pl.BlockSpec((F1, NC), lambda i: (0, 0)),
            pl.BlockSpec((1, NC), lambda i: (0, 0)),
            pl.BlockSpec((1, BB, 28, 28), lambda i: (i, 0, 0, 0)),
        ],
        out_specs=pl.BlockSpec((1, BB, NC), lambda i: (i, 0, 0)),
        compiler_params=pltpu.CompilerParams(
            dimension_semantics=("parallel",),
            vmem_limit_bytes=64 * 1024 * 1024),
    )(w1d, b1t, w2d, b2t, f1w, fc1_b, fc2_w, fc2_b, x_b)

    return out.reshape(nblk * BB, NC)[:n]


# speedup vs baseline: 2.3186x; 1.7350x over previous
import jax
import jax.numpy as jnp
from jax import lax
from jax.experimental import pallas as pl
from jax.experimental.pallas import tpu as pltpu

BB = 1024
KS = 5
C1, C2 = 4, 8
PH1, PW1 = 12, 12
PH2, PW2 = 4, 4
NF1 = PH1 * PW1 * C1
NF2 = PH2 * PW2 * C2
F1 = 32
NC = 10
PARITIES = ((0, 0), (0, 1), (1, 0), (1, 1))


def _fused_kernel(b1s_ref, b1t_ref, b2s_ref, b2t_ref, f1w_ref, f1b_ref,
                  f2w_ref, f2b_ref, x_ref, o_ref, w1t_ref, w2t_ref):
    base1 = b1s_ref[...]
    for i, (d, e) in enumerate(PARITIES):
        for p in range(PH1):
            for q in range(PW1):
                sh = (2 * p + d) * 28 + (2 * q + e)
                r = (p * PW1 + q) * C1
                w1t_ref[i, r:r + C1, :] = (
                    base1 if sh == 0 else jnp.roll(base1, sh, axis=1))
    base2 = b2s_ref[...]
    for i, (d, e) in enumerate(PARITIES):
        for p in range(PH2):
            for q in range(PW2):
                sh = ((2 * p + d) * PW1 + (2 * q + e)) * C1
                r = (p * PW2 + q) * C2
                w2t_ref[i, r:r + C2, :] = (
                    base2 if sh == 0 else jnp.roll(base2, sh, axis=1))

    x = x_ref[0].astype(jnp.bfloat16).reshape(BB, 784)
    dnums = (((1,), (1,)), ((), ()))
    o1 = [lax.dot_general(x, w1t_ref[i], dnums,
                          preferred_element_type=jnp.float32)
          for i in range(4)]
    p1 = jnp.maximum(jnp.maximum(o1[0], o1[1]), jnp.maximum(o1[2], o1[3]))
    p1 = jnp.maximum(p1 + b1t_ref[...], 0.0).astype(jnp.bfloat16)

    o2 = [lax.dot_general(p1, w2t_ref[i], dnums,
                          preferred_element_type=jnp.float32)
          for i in range(4)]
    p2 = jnp.maximum(jnp.maximum(o2[0], o2[1]), jnp.maximum(o2[2], o2[3]))
    p2 = jnp.maximum(p2 + b2t_ref[...], 0.0)

    y1 = jnp.dot(p2, f1w_ref[...], preferred_element_type=jnp.float32)
    y1 = jnp.maximum(y1 + f1b_ref[...], 0.0)
    logits = jnp.dot(y1, f2w_ref[...],
                     preferred_element_type=jnp.float32) + f2b_ref[...]
    z = logits - jnp.max(logits, axis=1, keepdims=True)
    lse = jnp.log(jnp.sum(jnp.exp(z), axis=1, keepdims=True))
    o_ref[0] = z - lse


def kernel(conv1_w, conv1_b, conv2_w, conv2_b, fc1_w, fc1_b, fc2_w, fc2_b, x):
    n = x.shape[0]
    pad = (-n) % BB
    if pad:
        x = jnp.concatenate(
            [x, jnp.zeros((pad,) + x.shape[1:], x.dtype)], axis=0)
    nblk = x.shape[0] // BB
    x_b = x.reshape(nblk, BB, 28, 28)

    b1s = jnp.pad(conv1_w.reshape(KS, KS, C1).transpose(2, 0, 1),
                  ((0, 0), (0, 28 - KS), (0, 28 - KS))
                  ).reshape(C1, 784).astype(jnp.bfloat16)
    b2s = jnp.pad(conv2_w.reshape(KS, KS, C1, C2).transpose(3, 0, 1, 2),
                  ((0, 0), (0, PH1 - KS), (0, PW1 - KS), (0, 0))
                  ).reshape(C2, NF1).astype(jnp.bfloat16)
    b1t = jnp.tile(conv1_b.reshape(1, C1), (1, PH1 * PW1))
    b2t = jnp.tile(conv2_b.reshape(1, C2), (1, PH2 * PW2))
    f1w = fc1_w.reshape(NF2, F1)

    out = pl.pallas_call(
        _fused_kernel,
        out_shape=jax.ShapeDtypeStruct((nblk, BB, NC), jnp.float32),
        grid=(nblk,),
        in_specs=[
            pl.BlockSpec((C1, 784), lambda i: (0, 0)),
            pl.BlockSpec((1, NF1), lambda i: (0, 0)),
            pl.BlockSpec((C2, NF1), lambda i: (0, 0)),
            pl.BlockSpec((1, NF2), lambda i: (0, 0)),
            pl.BlockSpec((NF2, F1), lambda i: (0, 0)),
            pl.BlockSpec((1, F1), lambda i: (0, 0)),
            pl.BlockSpec((F1, NC), lambda i: (0, 0)),
            pl.BlockSpec((1, NC), lambda i: (0, 0)),
            pl.BlockSpec((1, BB, 28, 28), lambda i: (i, 0, 0, 0)),
        ],
        out_specs=pl.BlockSpec((1, BB, NC), lambda i: (i, 0, 0)),
        scratch_shapes=[
            pltpu.VMEM((4, NF1, 784), jnp.bfloat16),
            pltpu.VMEM((4, NF2, NF1), jnp.bfloat16),
        ],
        compiler_params=pltpu.CompilerParams(
            dimension_semantics=("parallel",),
            vmem_limit_bytes=64 * 1024 * 1024),
    )(b1s, b1t, b2s, b2t, f1w, fc1_b, fc2_w, fc2_b, x_b)

    return out.reshape(nblk * BB, NC)[:n]


# speedup vs baseline: 2.5270x; 1.0899x over previous
import jax
import jax.numpy as jnp
from jax import lax
from jax.experimental import pallas as pl
from jax.experimental.pallas import tpu as pltpu

BB = 1024
KS = 5
C1, C2 = 4, 8
PH1, PW1 = 12, 12
PH2, PW2 = 4, 4
NF1 = PH1 * PW1 * C1
NF2 = PH2 * PW2 * C2
F1 = 32
NC = 10
PARITIES = ((0, 0), (0, 1), (1, 0), (1, 1))


def _fused_kernel(b1s_ref, b1t_ref, b2s_ref, b2t_ref, f1w_ref, f1b_ref,
                  f2w_ref, f2b_ref, x_ref, o_ref, w1t_ref, w2t_ref):
    @pl.when(pl.program_id(0) == 0)
    def _gen_weights():
        base1 = b1s_ref[...]
        for i, (d, e) in enumerate(PARITIES):
            for p in range(PH1):
                for q in range(PW1):
                    sh = (2 * p + d) * 28 + (2 * q + e)
                    r = i * NF1 + (p * PW1 + q) * C1
                    w1t_ref[r:r + C1, :] = (
                        base1 if sh == 0 else jnp.roll(base1, sh, axis=1))
        base2 = b2s_ref[...]
        for i, (d, e) in enumerate(PARITIES):
            for p in range(PH2):
                for q in range(PW2):
                    sh = ((2 * p + d) * PW1 + (2 * q + e)) * C1
                    r = i * NF2 + (p * PW2 + q) * C2
                    w2t_ref[r:r + C2, :] = (
                        base2 if sh == 0 else jnp.roll(base2, sh, axis=1))

    x = x_ref[0].astype(jnp.bfloat16).reshape(BB, 784)
    dnums = (((1,), (1,)), ((), ()))
    o1 = lax.dot_general(x, w1t_ref[...], dnums,
                         preferred_element_type=jnp.float32)
    p1 = jnp.maximum(jnp.maximum(o1[:, :NF1], o1[:, NF1:2 * NF1]),
                     jnp.maximum(o1[:, 2 * NF1:3 * NF1], o1[:, 3 * NF1:]))
    p1 = jnp.maximum(p1 + b1t_ref[...], 0.0).astype(jnp.bfloat16)

    o2 = lax.dot_general(p1, w2t_ref[...], dnums,
                         preferred_element_type=jnp.float32)
    p2 = jnp.maximum(jnp.maximum(o2[:, :NF2], o2[:, NF2:2 * NF2]),
                     jnp.maximum(o2[:, 2 * NF2:3 * NF2], o2[:, 3 * NF2:]))
    p2 = jnp.maximum(p2 + b2t_ref[...], 0.0)

    y1 = jnp.dot(p2, f1w_ref[...], preferred_element_type=jnp.float32)
    y1 = jnp.maximum(y1 + f1b_ref[...], 0.0)
    logits = jnp.dot(y1, f2w_ref[...],
                     preferred_element_type=jnp.float32) + f2b_ref[...]
    z = logits - jnp.max(logits, axis=1, keepdims=True)
    lse = jnp.log(jnp.sum(jnp.exp(z), axis=1, keepdims=True))
    o_ref[0] = z - lse


def kernel(conv1_w, conv1_b, conv2_w, conv2_b, fc1_w, fc1_b, fc2_w, fc2_b, x):
    n = x.shape[0]
    pad = (-n) % BB
    if pad:
        x = jnp.concatenate(
            [x, jnp.zeros((pad,) + x.shape[1:], x.dtype)], axis=0)
    nblk = x.shape[0] // BB
    x_b = x.reshape(nblk, BB, 28, 28)

    b1s = jnp.pad(conv1_w.reshape(KS, KS, C1).transpose(2, 0, 1),
                  ((0, 0), (0, 28 - KS), (0, 28 - KS))
                  ).reshape(C1, 784).astype(jnp.bfloat16)
    b2s = jnp.pad(conv2_w.reshape(KS, KS, C1, C2).transpose(3, 0, 1, 2),
                  ((0, 0), (0, PH1 - KS), (0, PW1 - KS), (0, 0))
                  ).reshape(C2, NF1).astype(jnp.bfloat16)
    b1t = jnp.tile(conv1_b.reshape(1, C1), (1, PH1 * PW1))
    b2t = jnp.tile(conv2_b.reshape(1, C2), (1, PH2 * PW2))
    f1w = fc1_w.reshape(NF2, F1)

    out = pl.pallas_call(
        _fused_kernel,
        out_shape=jax.ShapeDtypeStruct((nblk, BB, NC), jnp.float32),
        grid=(nblk,),
        in_specs=[
            pl.BlockSpec((C1, 784), lambda i: (0, 0)),
            pl.BlockSpec((1, NF1), lambda i: (0, 0)),
            pl.BlockSpec((C2, NF1), lambda i: (0, 0)),
            pl.BlockSpec((1, NF2), lambda i: (0, 0)),
            pl.BlockSpec((NF2, F1), lambda i: (0, 0)),
            pl.BlockSpec((1, F1), lambda i: (0, 0)),
            pl.BlockSpec((F1, NC), lambda i: (0, 0)),
            pl.BlockSpec((1, NC), lambda i: (0, 0)),
            pl.BlockSpec((1, BB, 28, 28), lambda i: (i, 0, 0, 0)),
        ],
        out_specs=pl.BlockSpec((1, BB, NC), lambda i: (i, 0, 0)),
        scratch_shapes=[
            pltpu.VMEM((4 * NF1, 784), jnp.bfloat16),
            pltpu.VMEM((4 * NF2, NF1), jnp.bfloat16),
        ],
        compiler_params=pltpu.CompilerParams(
            dimension_semantics=("arbitrary",),
            vmem_limit_bytes=64 * 1024 * 1024),
    )(b1s, b1t, b2s, b2t, f1w, fc1_b, fc2_w, fc2_b, x_b)

    return out.reshape(nblk * BB, NC)[:n]
